# baseline (speedup 1.0000x reference)
import dataclasses

import jax
import jax.numpy as jnp
from jax import lax
from jax.experimental import pallas as pl
from jax.experimental.pallas import tpu as pltpu
from jax.experimental.pallas import tpu_sc as plsc

D = 64
L = 16
NC = 2
NS = 16
NW = NC * NS

P = 512
CHUNK = 256
NBUF = 2


def _w2_kernel(v):
    def body(in_ref, out_ref):
        out_ref[:, 0:D] = jnp.swapaxes(in_ref[:, 0:P], 0, 1) * 8.0
        out_ref[:, D : 2 * D] = jnp.swapaxes(in_ref[:, P : 2 * P], 0, 1) * 8.0

    g = pl.cdiv(v, 2 * P)
    return pl.pallas_call(
        body,
        grid=(g,),
        in_specs=[pl.BlockSpec((D, 2 * P), lambda i: (0, i))],
        out_specs=pl.BlockSpec((P, 2 * D), lambda i: (i, 0)),
        out_shape=jax.ShapeDtypeStruct((g * P, 2 * D), jnp.float32),
    )


def _emb_kernel(n_total: int, seq: int, batch: int):
    b_per_w = n_total // NW
    steps = b_per_w // CHUNK
    assert n_total == NW * CHUNK * steps and steps % NBUF == 0
    assert batch % CHUNK == 0 and b_per_w % CHUNK == 0
    n_m = batch // 128
    cm = CHUNK // 128
    mesh = plsc.VectorSubcoreMesh(core_axis_name="c", subcore_axis_name="s")

    @pl.kernel(
        out_type=jax.ShapeDtypeStruct((seq, D // 8, n_m, 8, 128), jnp.float32),
        mesh=mesh,
        compiler_params=dataclasses.replace(
            pltpu.CompilerParams(use_tc_tiling_on_sc=False),
            needs_layout_passes=False,
        ),
        scratch_types=[
            pltpu.VMEM((b_per_w,), jnp.int32),
            pltpu.VMEM((NBUF, CHUNK, 2 * D), jnp.float32),
            pltpu.VMEM((NBUF, CHUNK), jnp.int32),
            pltpu.VMEM((NBUF, D // 8, cm, 8, 128), jnp.float32),
        ]
        + [pltpu.SemaphoreType.DMA] * (3 * NBUF),
    )
    def k(idx2_hbm, off_hbm, w2_hbm, out_hbm, idx2_v, blk_v, off_v, t_v, *sems):
        sg = sems[:NBUF]
        sf = sems[NBUF : 2 * NBUF]
        so = sems[2 * NBUF :]
        wid = lax.axis_index("s") * NC + lax.axis_index("c")
        base = wid * b_per_w
        pltpu.sync_copy(idx2_hbm.at[pl.ds(base, b_per_w)], idx2_v)

        def g_start(i, b):
            pltpu.async_copy(
                w2_hbm.at[idx2_v.at[pl.ds(i * CHUNK, CHUNK)]], blk_v.at[b], sg[b]
            )
            pltpu.async_copy(
                off_hbm.at[pl.ds(base + i * CHUNK, CHUNK)], off_v.at[b], sf[b]
            )

        def g_wait(i, b):
            pltpu.make_async_copy(
                w2_hbm.at[idx2_v.at[pl.ds(i * CHUNK, CHUNK)]], blk_v.at[b], sg[b]
            ).wait()
            pltpu.make_async_copy(
                off_hbm.at[pl.ds(base + i * CHUNK, CHUNK)], off_v.at[b], sf[b]
            ).wait()

        lb = batch.bit_length() - 1

        def o_dst(i, dg):
            j0 = base + i * CHUNK
            s = lax.shift_right_logical(j0, lb)
            m0 = lax.shift_right_logical(jnp.bitwise_and(j0, batch - 1), 7)
            return out_hbm.at[s, dg, pl.ds(m0, cm)]

        def o_start(i, b):
            for dg in range(D // 8):
                pltpu.async_copy(t_v.at[b, dg], o_dst(i, dg), so[b])

        def o_wait(i, b):
            for dg in range(D // 8):
                pltpu.make_async_copy(t_v.at[b, dg], o_dst(i, dg), so[b]).wait()

        iota = lax.iota(jnp.int32, L)

        def tec_chunk(b):
            rowvs = [iota + (bg * L) for bg in range(CHUNK // L)]
            parvs = [off_v[b, pl.ds(bg * L, L)] for bg in range(CHUNK // L)]

            @pl.loop(0, D)
            def _(d):
                dg = lax.shift_right_logical(d, 3)
                dr = jnp.bitwise_and(d, 7)
                for bg in range(CHUNK // L):
                    colv = parvs[bg] + d
                    val = plsc.load_gather(blk_v.at[b], [rowvs[bg], colv])
                    t_v[b, dg, bg // 8, dr, pl.ds((bg % 8) * L, L)] = val

        g_start(0, 0)
        g_start(1, 1)

        @pl.loop(0, steps, step=NBUF)
        def _(c):
            for u in range(NBUF):
                b = u
                j = c + u
                g_wait(j, b)

                @pl.when(j >= NBUF)
                def _(j=j, b=b):
                    o_wait(j - NBUF, b)

                tec_chunk(b)
                o_start(j, b)

                @pl.when(j + NBUF < steps)
                def _(j=j, b=b):
                    g_start(j + NBUF, b)

        o_wait(steps - 2, 0)
        o_wait(steps - 1, 1)

    return k


def kernel(x, W):
    batch, seq = x.shape
    v = W.shape[0]
    W2 = _w2_kernel(v)(W.T)
    idx = jnp.transpose(x).reshape(-1).astype(jnp.int32)
    lp = P.bit_length() - 1
    j = ((idx >> (lp + 1)) << lp) | (idx & (P - 1))
    off = ((idx >> lp) & 1) << 6
    out5 = _emb_kernel(idx.shape[0], seq, batch)(j, off, W2)
    t = jnp.transpose(out5, (2, 4, 0, 1, 3))
    return t.reshape(batch, seq, D)


# speedup vs baseline: 2.5240x; 2.5240x over previous
import dataclasses

import jax
import jax.numpy as jnp
from jax import lax
from jax.experimental import pallas as pl
from jax.experimental.pallas import tpu as pltpu
from jax.experimental.pallas import tpu_sc as plsc

D = 64
L = 16
NC = 2
NS = 16
NW = NC * NS

P = 512
CHUNK = 256
NBUF = 2


def _w2_kernel(v):
    gb = 4

    def body(in_ref, out_ref):
        for q in range(gb):
            out_ref[q * P : (q + 1) * P, 0:D] = (
                jnp.swapaxes(in_ref[:, 2 * q * P : (2 * q + 1) * P], 0, 1) * 8.0
            )
            out_ref[q * P : (q + 1) * P, D : 2 * D] = (
                jnp.swapaxes(in_ref[:, (2 * q + 1) * P : (2 * q + 2) * P], 0, 1) * 8.0
            )

    g = pl.cdiv(v, 2 * gb * P)
    return pl.pallas_call(
        body,
        grid=(g,),
        in_specs=[pl.BlockSpec((D, 2 * gb * P), lambda i: (0, i))],
        out_specs=pl.BlockSpec((gb * P, 2 * D), lambda i: (i, 0)),
        out_shape=jax.ShapeDtypeStruct((g * gb * P, 2 * D), jnp.float32),
    )


def _emb_kernel(n_total: int, seq: int, batch: int):
    b_per_w = n_total // NW
    steps = b_per_w // CHUNK
    assert n_total == NW * CHUNK * steps and steps % NBUF == 0
    assert batch % CHUNK == 0 and b_per_w % CHUNK == 0
    n_m = batch // 128
    cm = CHUNK // 128
    mesh = plsc.VectorSubcoreMesh(core_axis_name="c", subcore_axis_name="s")

    @pl.kernel(
        out_type=jax.ShapeDtypeStruct((seq, D // 8, n_m, 8, 128), jnp.float32),
        mesh=mesh,
        compiler_params=dataclasses.replace(
            pltpu.CompilerParams(use_tc_tiling_on_sc=False),
            needs_layout_passes=False,
        ),
        scratch_types=[
            pltpu.VMEM((b_per_w,), jnp.int32),
            pltpu.VMEM((NBUF, CHUNK, 2 * D), jnp.float32),
            pltpu.VMEM((NBUF, CHUNK), jnp.int32),
            pltpu.VMEM((NBUF, D // 8, cm, 8, 128), jnp.float32),
        ]
        + [pltpu.SemaphoreType.DMA] * (3 * NBUF),
    )
    def k(idx2_hbm, off_hbm, w2_hbm, out_hbm, idx2_v, blk_v, off_v, t_v, *sems):
        sg = sems[:NBUF]
        sf = sems[NBUF : 2 * NBUF]
        so = sems[2 * NBUF :]
        wid = lax.axis_index("s") * NC + lax.axis_index("c")
        base = wid * b_per_w
        pltpu.sync_copy(idx2_hbm.at[pl.ds(base, b_per_w)], idx2_v)

        def g_start(i, b):
            pltpu.async_copy(
                w2_hbm.at[idx2_v.at[pl.ds(i * CHUNK, CHUNK)]], blk_v.at[b], sg[b]
            )
            pltpu.async_copy(
                off_hbm.at[pl.ds(base + i * CHUNK, CHUNK)], off_v.at[b], sf[b]
            )

        def g_wait(i, b):
            pltpu.make_async_copy(
                w2_hbm.at[idx2_v.at[pl.ds(i * CHUNK, CHUNK)]], blk_v.at[b], sg[b]
            ).wait()
            pltpu.make_async_copy(
                off_hbm.at[pl.ds(base + i * CHUNK, CHUNK)], off_v.at[b], sf[b]
            ).wait()

        lb = batch.bit_length() - 1

        def o_dst(i, dg):
            j0 = base + i * CHUNK
            s = lax.shift_right_logical(j0, lb)
            m0 = lax.shift_right_logical(jnp.bitwise_and(j0, batch - 1), 7)
            return out_hbm.at[s, dg, pl.ds(m0, cm)]

        def o_start(i, b):
            for dg in range(D // 8):
                pltpu.async_copy(t_v.at[b, dg], o_dst(i, dg), so[b])

        def o_wait(i, b):
            for dg in range(D // 8):
                pltpu.make_async_copy(t_v.at[b, dg], o_dst(i, dg), so[b]).wait()

        iota = lax.iota(jnp.int32, L)

        def tec_chunk(b):
            rowvs = [iota + (bg * L) for bg in range(CHUNK // L)]
            lanevs = [jnp.bitwise_and(r, 127) for r in rowvs[:8]]
            msplat = [jnp.full((L,), m, jnp.int32) for m in range(cm)]
            parvs = [off_v[b, pl.ds(bg * L, L)] for bg in range(CHUNK // L)]

            @pl.loop(0, D)
            def _(d):
                dpl = jnp.bitwise_and(iota + d, D - 1)
                dgv = lax.shift_right_logical(dpl, 3)
                drv = jnp.bitwise_and(dpl, 7)
                for bg in range(CHUNK // L):
                    colv = parvs[bg] + dpl
                    val = plsc.load_gather(blk_v.at[b], [rowvs[bg], colv])
                    plsc.store_scatter(
                        t_v.at[b],
                        [dgv, msplat[bg * L // 128], drv, lanevs[bg % 8]],
                        val,
                    )

        g_start(0, 0)
        g_start(1, 1)

        @pl.loop(0, steps, step=NBUF)
        def _(c):
            for u in range(NBUF):
                b = u
                j = c + u
                g_wait(j, b)

                @pl.when(j >= NBUF)
                def _(j=j, b=b):
                    o_wait(j - NBUF, b)

                tec_chunk(b)
                o_start(j, b)

                @pl.when(j + NBUF < steps)
                def _(j=j, b=b):
                    g_start(j + NBUF, b)

        o_wait(steps - 2, 0)
        o_wait(steps - 1, 1)

    return k


def kernel(x, W):
    batch, seq = x.shape
    v = W.shape[0]
    W2 = _w2_kernel(v)(W.T)
    idx = jnp.transpose(x).reshape(-1).astype(jnp.int32)
    lp = P.bit_length() - 1
    j = ((idx >> (lp + 1)) << lp) | (idx & (P - 1))
    off = ((idx >> lp) & 1) << 6
    out5 = _emb_kernel(idx.shape[0], seq, batch)(j, off, W2)
    t = jnp.transpose(out5, (2, 4, 0, 1, 3))
    return t.reshape(batch, seq, D)


# speedup vs baseline: 2.7779x; 1.1006x over previous
import dataclasses

import jax
import jax.numpy as jnp
from jax import lax
from jax.experimental import pallas as pl
from jax.experimental.pallas import tpu as pltpu
from jax.experimental.pallas import tpu_sc as plsc

D = 64
L = 16
NC = 2
NS = 16
NW = NC * NS

P = 512
CHUNK = 256
NBUF = 2


def _w2_kernel(v):
    gb = 8

    def body(in_ref, out_ref):
        for q in range(gb):
            out_ref[q * P : (q + 1) * P, 0:D] = (
                jnp.swapaxes(in_ref[:, 2 * q * P : (2 * q + 1) * P], 0, 1) * 8.0
            )
            out_ref[q * P : (q + 1) * P, D : 2 * D] = (
                jnp.swapaxes(in_ref[:, (2 * q + 1) * P : (2 * q + 2) * P], 0, 1) * 8.0
            )

    g = pl.cdiv(v, 2 * gb * P)
    return pl.pallas_call(
        body,
        grid=(g,),
        in_specs=[pl.BlockSpec((D, 2 * gb * P), lambda i: (0, i))],
        out_specs=pl.BlockSpec((gb * P, 2 * D), lambda i: (i, 0)),
        out_shape=jax.ShapeDtypeStruct((g * gb * P, 2 * D), jnp.float32),
    )


def _emb_kernel(n_total: int, seq: int, batch: int):
    b_per_w = n_total // NW
    steps = b_per_w // CHUNK
    assert n_total == NW * CHUNK * steps and steps % NBUF == 0
    assert batch % CHUNK == 0 and b_per_w % CHUNK == 0
    n_m = batch // 128
    cm = CHUNK // 128
    mesh = plsc.VectorSubcoreMesh(core_axis_name="c", subcore_axis_name="s")

    @pl.kernel(
        out_type=jax.ShapeDtypeStruct((seq, D // 8, n_m, 8, 128), jnp.float32),
        mesh=mesh,
        compiler_params=dataclasses.replace(
            pltpu.CompilerParams(use_tc_tiling_on_sc=False),
            needs_layout_passes=False,
        ),
        scratch_types=[
            pltpu.VMEM((b_per_w,), jnp.int32),
            pltpu.VMEM((NBUF, CHUNK, 2 * D), jnp.float32),
            pltpu.VMEM((NBUF, CHUNK), jnp.int32),
            pltpu.VMEM((NBUF, D // 8, cm, 8, 128), jnp.float32),
        ]
        + [pltpu.SemaphoreType.DMA] * (3 * NBUF),
    )
    def k(idx2_hbm, off_hbm, w2_hbm, out_hbm, idx2_v, blk_v, off_v, t_v, *sems):
        sg = sems[:NBUF]
        sf = sems[NBUF : 2 * NBUF]
        so = sems[2 * NBUF :]
        wid = lax.axis_index("s") * NC + lax.axis_index("c")
        base = wid * b_per_w
        pltpu.sync_copy(idx2_hbm.at[pl.ds(base, b_per_w)], idx2_v)

        def g_start(i, b):
            pltpu.async_copy(
                w2_hbm.at[idx2_v.at[pl.ds(i * CHUNK, CHUNK)]], blk_v.at[b], sg[b]
            )
            pltpu.async_copy(
                off_hbm.at[pl.ds(base + i * CHUNK, CHUNK)], off_v.at[b], sf[b]
            )

        def g_wait(i, b):
            pltpu.make_async_copy(
                w2_hbm.at[idx2_v.at[pl.ds(i * CHUNK, CHUNK)]], blk_v.at[b], sg[b]
            ).wait()
            pltpu.make_async_copy(
                off_hbm.at[pl.ds(base + i * CHUNK, CHUNK)], off_v.at[b], sf[b]
            ).wait()

        lb = batch.bit_length() - 1

        def o_dst(i, dg):
            j0 = base + i * CHUNK
            s = lax.shift_right_logical(j0, lb)
            m0 = lax.shift_right_logical(jnp.bitwise_and(j0, batch - 1), 7)
            return out_hbm.at[s, dg, pl.ds(m0, cm)]

        def o_start(i, b):
            for dg in range(D // 8):
                pltpu.async_copy(t_v.at[b, dg], o_dst(i, dg), so[b])

        def o_wait(i, b):
            for dg in range(D // 8):
                pltpu.make_async_copy(t_v.at[b, dg], o_dst(i, dg), so[b]).wait()

        iota = lax.iota(jnp.int32, L)

        def tec_chunk(b):
            rowvs = [iota + (bg * L) for bg in range(CHUNK // L)]
            lanevs = [jnp.bitwise_and(r, 127) for r in rowvs[:8]]
            msplat = [jnp.full((L,), m, jnp.int32) for m in range(cm)]
            parvs = [off_v[b, pl.ds(bg * L, L)] for bg in range(CHUNK // L)]

            @pl.loop(0, D)
            def _(d):
                dpl = jnp.bitwise_and(iota + d, D - 1)
                dgv = lax.shift_right_logical(dpl, 3)
                drv = jnp.bitwise_and(dpl, 7)
                for bg in range(CHUNK // L):
                    colv = parvs[bg] + dpl
                    val = plsc.load_gather(blk_v.at[b], [rowvs[bg], colv])
                    plsc.store_scatter(
                        t_v.at[b],
                        [dgv, msplat[bg * L // 128], drv, lanevs[bg % 8]],
                        val,
                    )

        g_start(0, 0)
        g_start(1, 1)

        @pl.loop(0, steps, step=NBUF)
        def _(c):
            for u in range(NBUF):
                b = u
                j = c + u
                g_wait(j, b)

                @pl.when(j >= NBUF)
                def _(j=j, b=b):
                    o_wait(j - NBUF, b)

                tec_chunk(b)
                o_start(j, b)

                @pl.when(j + NBUF < steps)
                def _(j=j, b=b):
                    g_start(j + NBUF, b)

        o_wait(steps - 2, 0)
        o_wait(steps - 1, 1)

    return k


def kernel(x, W):
    batch, seq = x.shape
    v = W.shape[0]
    W2 = _w2_kernel(v)(W.T)
    idx = jnp.transpose(x).reshape(-1).astype(jnp.int32)
    lp = P.bit_length() - 1
    j = ((idx >> (lp + 1)) << lp) | (idx & (P - 1))
    off = ((idx >> lp) & 1) << 6
    out5 = _emb_kernel(idx.shape[0], seq, batch)(j, off, W2)
    t = jnp.transpose(out5, (2, 4, 0, 1, 3))
    return t.reshape(batch, seq, D)


# speedup vs baseline: 2.9099x; 1.0475x over previous
import dataclasses

import jax
import jax.numpy as jnp
from jax import lax
from jax.experimental import pallas as pl
from jax.experimental.pallas import tpu as pltpu
from jax.experimental.pallas import tpu_sc as plsc

D = 64
L = 16
NC = 2
NS = 16
NW = NC * NS

P = 512
CHUNK = 256
NBUF = 2


def _w2_kernel(v):
    gb = 16

    def body(in_ref, out_ref):
        for q in range(gb):
            out_ref[q * P : (q + 1) * P, 0:D] = (
                jnp.swapaxes(in_ref[:, 2 * q * P : (2 * q + 1) * P], 0, 1) * 8.0
            )
            out_ref[q * P : (q + 1) * P, D : 2 * D] = (
                jnp.swapaxes(in_ref[:, (2 * q + 1) * P : (2 * q + 2) * P], 0, 1) * 8.0
            )

    g = pl.cdiv(v, 2 * gb * P)
    return pl.pallas_call(
        body,
        grid=(g,),
        in_specs=[pl.BlockSpec((D, 2 * gb * P), lambda i: (0, i))],
        out_specs=pl.BlockSpec((gb * P, 2 * D), lambda i: (i, 0)),
        out_shape=jax.ShapeDtypeStruct((g * gb * P, 2 * D), jnp.float32),
    )


def _emb_kernel(n_total: int, seq: int, batch: int):
    b_per_w = n_total // NW
    steps = b_per_w // CHUNK
    assert n_total == NW * CHUNK * steps and steps % NBUF == 0
    assert batch % CHUNK == 0 and b_per_w % CHUNK == 0
    n_m = batch // 128
    cm = CHUNK // 128
    mesh = plsc.VectorSubcoreMesh(core_axis_name="c", subcore_axis_name="s")

    @pl.kernel(
        out_type=jax.ShapeDtypeStruct((seq, D // 8, n_m, 8, 128), jnp.float32),
        mesh=mesh,
        compiler_params=dataclasses.replace(
            pltpu.CompilerParams(use_tc_tiling_on_sc=False),
            needs_layout_passes=False,
        ),
        scratch_types=[
            pltpu.VMEM((b_per_w,), jnp.int32),
            pltpu.VMEM((NBUF, CHUNK, 2 * D), jnp.float32),
            pltpu.VMEM((NBUF, CHUNK), jnp.int32),
            pltpu.VMEM((NBUF, D // 8, cm, 8, 128), jnp.float32),
        ]
        + [pltpu.SemaphoreType.DMA] * (3 * NBUF),
    )
    def k(idx2_hbm, off_hbm, w2_hbm, out_hbm, idx2_v, blk_v, off_v, t_v, *sems):
        sg = sems[:NBUF]
        sf = sems[NBUF : 2 * NBUF]
        so = sems[2 * NBUF :]
        wid = lax.axis_index("s") * NC + lax.axis_index("c")
        base = wid * b_per_w
        pltpu.sync_copy(idx2_hbm.at[pl.ds(base, b_per_w)], idx2_v)

        def g_start(i, b):
            pltpu.async_copy(
                w2_hbm.at[idx2_v.at[pl.ds(i * CHUNK, CHUNK)]], blk_v.at[b], sg[b]
            )
            pltpu.async_copy(
                off_hbm.at[pl.ds(base + i * CHUNK, CHUNK)], off_v.at[b], sf[b]
            )

        def g_wait(i, b):
            pltpu.make_async_copy(
                w2_hbm.at[idx2_v.at[pl.ds(i * CHUNK, CHUNK)]], blk_v.at[b], sg[b]
            ).wait()
            pltpu.make_async_copy(
                off_hbm.at[pl.ds(base + i * CHUNK, CHUNK)], off_v.at[b], sf[b]
            ).wait()

        lb = batch.bit_length() - 1

        def o_dst(i, dg):
            j0 = base + i * CHUNK
            s = lax.shift_right_logical(j0, lb)
            m0 = lax.shift_right_logical(jnp.bitwise_and(j0, batch - 1), 7)
            return out_hbm.at[s, dg, pl.ds(m0, cm)]

        def o_start(i, b):
            for dg in range(D // 8):
                pltpu.async_copy(t_v.at[b, dg], o_dst(i, dg), so[b])

        def o_wait(i, b):
            for dg in range(D // 8):
                pltpu.make_async_copy(t_v.at[b, dg], o_dst(i, dg), so[b]).wait()

        iota = lax.iota(jnp.int32, L)

        def tec_chunk(b):
            rowvs = [iota + (bg * L) for bg in range(CHUNK // L)]
            lanevs = [jnp.bitwise_and(r, 127) for r in rowvs[:8]]
            msplat = [jnp.full((L,), m, jnp.int32) for m in range(cm)]
            parvs = [off_v[b, pl.ds(bg * L, L)] for bg in range(CHUNK // L)]

            @pl.loop(0, D)
            def _(d):
                dpl = jnp.bitwise_and(iota + d, D - 1)
                dgv = lax.shift_right_logical(dpl, 3)
                drv = jnp.bitwise_and(dpl, 7)
                for bg in range(CHUNK // L):
                    colv = parvs[bg] + dpl
                    val = plsc.load_gather(blk_v.at[b], [rowvs[bg], colv])
                    plsc.store_scatter(
                        t_v.at[b],
                        [dgv, msplat[bg * L // 128], drv, lanevs[bg % 8]],
                        val,
                    )

        g_start(0, 0)
        g_start(1, 1)

        @pl.loop(0, steps, step=NBUF)
        def _(c):
            for u in range(NBUF):
                b = u
                j = c + u
                g_wait(j, b)

                @pl.when(j >= NBUF)
                def _(j=j, b=b):
                    o_wait(j - NBUF, b)

                tec_chunk(b)
                o_start(j, b)

                @pl.when(j + NBUF < steps)
                def _(j=j, b=b):
                    g_start(j + NBUF, b)

        o_wait(steps - 2, 0)
        o_wait(steps - 1, 1)

    return k


def kernel(x, W):
    batch, seq = x.shape
    v = W.shape[0]
    W2 = _w2_kernel(v)(W.T)
    idx = jnp.transpose(x).reshape(-1).astype(jnp.int32)
    lp = P.bit_length() - 1
    j = ((idx >> (lp + 1)) << lp) | (idx & (P - 1))
    off = ((idx >> lp) & 1) << 6
    out5 = _emb_kernel(idx.shape[0], seq, batch)(j, off, W2)
    t = jnp.transpose(out5, (2, 4, 0, 1, 3))
    return t.reshape(batch, seq, D)


# speedup vs baseline: 2.9829x; 1.0251x over previous
import dataclasses

import jax
import jax.numpy as jnp
from jax import lax
from jax.experimental import pallas as pl
from jax.experimental.pallas import tpu as pltpu
from jax.experimental.pallas import tpu_sc as plsc

D = 64
L = 16
NC = 2
NS = 16
NW = NC * NS

P = 512
CHUNK = 256
NBUF = 2


def _w2_kernel(v):
    gb = 32

    def body(in_ref, out_ref):
        for q in range(gb):
            out_ref[q * P : (q + 1) * P, 0:D] = (
                jnp.swapaxes(in_ref[:, 2 * q * P : (2 * q + 1) * P], 0, 1) * 8.0
            )
            out_ref[q * P : (q + 1) * P, D : 2 * D] = (
                jnp.swapaxes(in_ref[:, (2 * q + 1) * P : (2 * q + 2) * P], 0, 1) * 8.0
            )

    g = pl.cdiv(v, 2 * gb * P)
    return pl.pallas_call(
        body,
        grid=(g,),
        in_specs=[pl.BlockSpec((D, 2 * gb * P), lambda i: (0, i))],
        out_specs=pl.BlockSpec((gb * P, 2 * D), lambda i: (i, 0)),
        out_shape=jax.ShapeDtypeStruct((g * gb * P, 2 * D), jnp.float32),
    )


def _emb_kernel(n_total: int, seq: int, batch: int):
    b_per_w = n_total // NW
    steps = b_per_w // CHUNK
    assert n_total == NW * CHUNK * steps and steps % NBUF == 0
    assert batch % CHUNK == 0 and b_per_w % CHUNK == 0
    n_m = batch // 128
    cm = CHUNK // 128
    mesh = plsc.VectorSubcoreMesh(core_axis_name="c", subcore_axis_name="s")

    @pl.kernel(
        out_type=jax.ShapeDtypeStruct((seq, D // 8, n_m, 8, 128), jnp.float32),
        mesh=mesh,
        compiler_params=dataclasses.replace(
            pltpu.CompilerParams(use_tc_tiling_on_sc=False),
            needs_layout_passes=False,
        ),
        scratch_types=[
            pltpu.VMEM((b_per_w,), jnp.int32),
            pltpu.VMEM((NBUF, CHUNK, 2 * D), jnp.float32),
            pltpu.VMEM((NBUF, CHUNK), jnp.int32),
            pltpu.VMEM((NBUF, D // 8, cm, 8, 128), jnp.float32),
        ]
        + [pltpu.SemaphoreType.DMA] * (3 * NBUF),
    )
    def k(idx2_hbm, off_hbm, w2_hbm, out_hbm, idx2_v, blk_v, off_v, t_v, *sems):
        sg = sems[:NBUF]
        sf = sems[NBUF : 2 * NBUF]
        so = sems[2 * NBUF :]
        wid = lax.axis_index("s") * NC + lax.axis_index("c")
        base = wid * b_per_w
        pltpu.sync_copy(idx2_hbm.at[pl.ds(base, b_per_w)], idx2_v)

        def g_start(i, b):
            pltpu.async_copy(
                w2_hbm.at[idx2_v.at[pl.ds(i * CHUNK, CHUNK)]], blk_v.at[b], sg[b]
            )
            pltpu.async_copy(
                off_hbm.at[pl.ds(base + i * CHUNK, CHUNK)], off_v.at[b], sf[b]
            )

        def g_wait(i, b):
            pltpu.make_async_copy(
                w2_hbm.at[idx2_v.at[pl.ds(i * CHUNK, CHUNK)]], blk_v.at[b], sg[b]
            ).wait()
            pltpu.make_async_copy(
                off_hbm.at[pl.ds(base + i * CHUNK, CHUNK)], off_v.at[b], sf[b]
            ).wait()

        lb = batch.bit_length() - 1

        def o_dst(i, dg):
            j0 = base + i * CHUNK
            s = lax.shift_right_logical(j0, lb)
            m0 = lax.shift_right_logical(jnp.bitwise_and(j0, batch - 1), 7)
            return out_hbm.at[s, dg, pl.ds(m0, cm)]

        def o_start(i, b):
            for dg in range(D // 8):
                pltpu.async_copy(t_v.at[b, dg], o_dst(i, dg), so[b])

        def o_wait(i, b):
            for dg in range(D // 8):
                pltpu.make_async_copy(t_v.at[b, dg], o_dst(i, dg), so[b]).wait()

        iota = lax.iota(jnp.int32, L)

        def tec_chunk(b):
            rowvs = [iota + (bg * L) for bg in range(CHUNK // L)]
            lanevs = [jnp.bitwise_and(r, 127) for r in rowvs[:8]]
            msplat = [jnp.full((L,), m, jnp.int32) for m in range(cm)]
            parvs = [off_v[b, pl.ds(bg * L, L)] for bg in range(CHUNK // L)]

            @pl.loop(0, D)
            def _(d):
                dpl = jnp.bitwise_and(iota + d, D - 1)
                dgv = lax.shift_right_logical(dpl, 3)
                drv = jnp.bitwise_and(dpl, 7)
                for bg in range(CHUNK // L):
                    colv = parvs[bg] + dpl
                    val = plsc.load_gather(blk_v.at[b], [rowvs[bg], colv])
                    plsc.store_scatter(
                        t_v.at[b],
                        [dgv, msplat[bg * L // 128], drv, lanevs[bg % 8]],
                        val,
                    )

        g_start(0, 0)
        g_start(1, 1)

        @pl.loop(0, steps, step=NBUF)
        def _(c):
            for u in range(NBUF):
                b = u
                j = c + u
                g_wait(j, b)

                @pl.when(j >= NBUF)
                def _(j=j, b=b):
                    o_wait(j - NBUF, b)

                tec_chunk(b)
                o_start(j, b)

                @pl.when(j + NBUF < steps)
                def _(j=j, b=b):
                    g_start(j + NBUF, b)

        o_wait(steps - 2, 0)
        o_wait(steps - 1, 1)

    return k


def kernel(x, W):
    batch, seq = x.shape
    v = W.shape[0]
    W2 = _w2_kernel(v)(W.T)
    idx = jnp.transpose(x).reshape(-1).astype(jnp.int32)
    lp = P.bit_length() - 1
    j = ((idx >> (lp + 1)) << lp) | (idx & (P - 1))
    off = ((idx >> lp) & 1) << 6
    out5 = _emb_kernel(idx.shape[0], seq, batch)(j, off, W2)
    t = jnp.transpose(out5, (2, 4, 0, 1, 3))
    return t.reshape(batch, seq, D)


# speedup vs baseline: 2.9883x; 1.0018x over previous
import dataclasses

import jax
import jax.numpy as jnp
from jax import lax
from jax.experimental import pallas as pl
from jax.experimental.pallas import tpu as pltpu
from jax.experimental.pallas import tpu_sc as plsc

D = 64
L = 16
NC = 2
NS = 16
NW = NC * NS

P = 512
CHUNK = 256
NBUF = 2


def _w2_kernel(v):
    gb = 32

    def body(in_ref, out_ref):
        for q in range(gb):
            out_ref[q * P : (q + 1) * P, 0:D] = (
                jnp.swapaxes(in_ref[:, 2 * q * P : (2 * q + 1) * P], 0, 1) * 8.0
            )
            out_ref[q * P : (q + 1) * P, D : 2 * D] = (
                jnp.swapaxes(in_ref[:, (2 * q + 1) * P : (2 * q + 2) * P], 0, 1) * 8.0
            )

    g = pl.cdiv(v, 2 * gb * P)
    return pl.pallas_call(
        body,
        grid=(g,),
        in_specs=[pl.BlockSpec((D, 2 * gb * P), lambda i: (0, i))],
        out_specs=pl.BlockSpec((gb * P, 2 * D), lambda i: (i, 0)),
        out_shape=jax.ShapeDtypeStruct((g * gb * P, 2 * D), jnp.float32),
    )


def _emb_kernel(n_total: int, seq: int, batch: int):
    b_per_w = n_total // NW
    steps = b_per_w // CHUNK
    assert n_total == NW * CHUNK * steps and steps % NBUF == 0
    assert batch % CHUNK == 0 and b_per_w % CHUNK == 0
    n_m = batch // 128
    cm = CHUNK // 128
    mesh = plsc.VectorSubcoreMesh(core_axis_name="c", subcore_axis_name="s")

    @pl.kernel(
        out_type=jax.ShapeDtypeStruct((seq, D // 8, n_m, 8, 128), jnp.float32),
        mesh=mesh,
        compiler_params=dataclasses.replace(
            pltpu.CompilerParams(use_tc_tiling_on_sc=False),
            needs_layout_passes=False,
        ),
        scratch_types=[
            pltpu.VMEM((b_per_w,), jnp.int32),
            pltpu.VMEM((NBUF, CHUNK, 2 * D), jnp.float32),
            pltpu.VMEM((NBUF, CHUNK), jnp.int32),
            pltpu.VMEM((NBUF, D // 8, cm, 8, 128), jnp.float32),
        ]
        + [pltpu.SemaphoreType.DMA] * (3 * NBUF),
    )
    def k(idx2_hbm, off_hbm, w2_hbm, out_hbm, idx2_v, blk_v, off_v, t_v, *sems):
        sg = sems[:NBUF]
        sf = sems[NBUF : 2 * NBUF]
        so = sems[2 * NBUF :]
        wid = lax.axis_index("s") * NC + lax.axis_index("c")
        base = wid * b_per_w
        pltpu.sync_copy(idx2_hbm.at[pl.ds(base, b_per_w)], idx2_v)

        def g_start(i, b):
            pltpu.async_copy(
                w2_hbm.at[idx2_v.at[pl.ds(i * CHUNK, CHUNK)]], blk_v.at[b], sg[b]
            )
            pltpu.async_copy(
                off_hbm.at[pl.ds(base + i * CHUNK, CHUNK)], off_v.at[b], sf[b]
            )

        def g_wait(i, b):
            pltpu.make_async_copy(
                w2_hbm.at[idx2_v.at[pl.ds(i * CHUNK, CHUNK)]], blk_v.at[b], sg[b]
            ).wait()
            pltpu.make_async_copy(
                off_hbm.at[pl.ds(base + i * CHUNK, CHUNK)], off_v.at[b], sf[b]
            ).wait()

        lb = batch.bit_length() - 1

        def o_dst(i):
            j0 = base + i * CHUNK
            s = lax.shift_right_logical(j0, lb)
            m0 = lax.shift_right_logical(jnp.bitwise_and(j0, batch - 1), 7)
            return out_hbm.at[s, :, pl.ds(m0, cm)]

        def o_start(i, b):
            pltpu.async_copy(t_v.at[b], o_dst(i), so[b])

        def o_wait(i, b):
            pltpu.make_async_copy(t_v.at[b], o_dst(i), so[b]).wait()

        iota = lax.iota(jnp.int32, L)

        def tec_chunk(b):
            rowvs = [iota + (bg * L) for bg in range(CHUNK // L)]
            lanevs = [jnp.bitwise_and(r, 127) for r in rowvs[:8]]
            msplat = [jnp.full((L,), m, jnp.int32) for m in range(cm)]
            parvs = [off_v[b, pl.ds(bg * L, L)] for bg in range(CHUNK // L)]

            @pl.loop(0, D)
            def _(d):
                dpl = jnp.bitwise_and(iota + d, D - 1)
                dgv = lax.shift_right_logical(dpl, 3)
                drv = jnp.bitwise_and(dpl, 7)
                for bg in range(CHUNK // L):
                    colv = parvs[bg] + dpl
                    val = plsc.load_gather(blk_v.at[b], [rowvs[bg], colv])
                    plsc.store_scatter(
                        t_v.at[b],
                        [dgv, msplat[bg * L // 128], drv, lanevs[bg % 8]],
                        val,
                    )

        g_start(0, 0)
        g_start(1, 1)

        @pl.loop(0, steps, step=NBUF)
        def _(c):
            for u in range(NBUF):
                b = u
                j = c + u
                g_wait(j, b)

                @pl.when(j >= NBUF)
                def _(j=j, b=b):
                    o_wait(j - NBUF, b)

                tec_chunk(b)
                o_start(j, b)

                @pl.when(j + NBUF < steps)
                def _(j=j, b=b):
                    g_start(j + NBUF, b)

        o_wait(steps - 2, 0)
        o_wait(steps - 1, 1)

    return k


def kernel(x, W):
    batch, seq = x.shape
    v = W.shape[0]
    W2 = _w2_kernel(v)(W.T)
    idx = jnp.transpose(x).reshape(-1).astype(jnp.int32)
    lp = P.bit_length() - 1
    j = ((idx >> (lp + 1)) << lp) | (idx & (P - 1))
    off = ((idx >> lp) & 1) << 6
    out5 = _emb_kernel(idx.shape[0], seq, batch)(j, off, W2)
    t = jnp.transpose(out5, (2, 4, 0, 1, 3))
    return t.reshape(batch, seq, D)


# speedup vs baseline: 4.2484x; 1.4217x over previous
import dataclasses

import jax
import jax.numpy as jnp
from jax import lax
from jax.experimental import pallas as pl
from jax.experimental.pallas import tpu as pltpu
from jax.experimental.pallas import tpu_sc as plsc

D = 64
L = 16
NC = 2
NS = 16
NW = NC * NS

P = 512
CHUNK = 256
NBUF = 2


def _w2_kernel(v):
    gb = 32

    def body(in_ref, out_ref):
        for q in range(gb):
            out_ref[q * P : (q + 1) * P, 0:D] = (
                jnp.swapaxes(in_ref[:, 2 * q * P : (2 * q + 1) * P], 0, 1) * 8.0
            )
            out_ref[q * P : (q + 1) * P, D : 2 * D] = (
                jnp.swapaxes(in_ref[:, (2 * q + 1) * P : (2 * q + 2) * P], 0, 1) * 8.0
            )

    g = pl.cdiv(v, 2 * gb * P)
    return pl.pallas_call(
        body,
        grid=(g,),
        in_specs=[pl.BlockSpec((D, 2 * gb * P), lambda i: (0, i))],
        out_specs=pl.BlockSpec((gb * P, 2 * D), lambda i: (i, 0)),
        out_shape=jax.ShapeDtypeStruct((g * gb * P, 2 * D), jnp.float32),
    )


def _emb_kernel(n_total: int, seq: int, batch: int):
    b_per_w = n_total // NW
    steps = b_per_w // CHUNK
    assert n_total == NW * CHUNK * steps and steps % NBUF == 0
    assert batch % CHUNK == 0 and b_per_w % CHUNK == 0
    n_m = batch // 128
    cm = CHUNK // 128
    mesh = plsc.VectorSubcoreMesh(core_axis_name="c", subcore_axis_name="s")

    @pl.kernel(
        out_type=jax.ShapeDtypeStruct((seq, D // 8, n_m, 8, 128), jnp.float32),
        mesh=mesh,
        compiler_params=dataclasses.replace(
            pltpu.CompilerParams(use_tc_tiling_on_sc=False),
            needs_layout_passes=False,
        ),
        scratch_types=[
            pltpu.VMEM((b_per_w,), jnp.int32),
            pltpu.VMEM((NBUF, CHUNK, 2 * D), jnp.float32),
            pltpu.VMEM((NBUF, CHUNK), jnp.int32),
            pltpu.VMEM((NBUF, D // 8, cm, 8, 128), jnp.float32),
        ]
        + [pltpu.SemaphoreType.DMA] * (3 * NBUF),
    )
    def k(idx2_hbm, off_hbm, w2_hbm, out_hbm, idx2_v, blk_v, off_v, t_v, *sems):
        sg = sems[:NBUF]
        sf = sems[NBUF : 2 * NBUF]
        so = sems[2 * NBUF :]
        wid = lax.axis_index("s") * NC + lax.axis_index("c")
        base = wid * b_per_w
        pltpu.sync_copy(idx2_hbm.at[pl.ds(base, b_per_w)], idx2_v)

        def g_start(i, b):
            pltpu.async_copy(
                w2_hbm.at[idx2_v.at[pl.ds(i * CHUNK, CHUNK)]], blk_v.at[b], sg[b]
            )
            pltpu.async_copy(
                off_hbm.at[pl.ds(base + i * CHUNK, CHUNK)], off_v.at[b], sf[b]
            )

        def g_wait(i, b):
            pltpu.make_async_copy(
                w2_hbm.at[idx2_v.at[pl.ds(i * CHUNK, CHUNK)]], blk_v.at[b], sg[b]
            ).wait()
            pltpu.make_async_copy(
                off_hbm.at[pl.ds(base + i * CHUNK, CHUNK)], off_v.at[b], sf[b]
            ).wait()

        lb = batch.bit_length() - 1

        def o_dst(i):
            j0 = base + i * CHUNK
            s = lax.shift_right_logical(j0, lb)
            m0 = lax.shift_right_logical(jnp.bitwise_and(j0, batch - 1), 7)
            return out_hbm.at[s, :, pl.ds(m0, cm)]

        def o_start(i, b):
            pltpu.async_copy(t_v.at[b], o_dst(i), so[b])

        def o_wait(i, b):
            pltpu.make_async_copy(t_v.at[b], o_dst(i), so[b]).wait()

        iota = lax.iota(jnp.int32, L)

        def tec_chunk(b):
            rowvs = [iota + (bg * L) for bg in range(CHUNK // L)]
            lanevs = [jnp.bitwise_and(r, 127) for r in rowvs[:8]]
            msplat = [jnp.full((L,), m, jnp.int32) for m in range(cm)]
            parvs = [off_v[b, pl.ds(bg * L, L)] for bg in range(CHUNK // L)]

            @plsc.parallel_loop(0, D, unroll=2)
            def _(d):
                dpl = jnp.bitwise_and(iota + d, D - 1)
                dgv = lax.shift_right_logical(dpl, 3)
                drv = jnp.bitwise_and(dpl, 7)
                for bg in range(CHUNK // L):
                    colv = parvs[bg] + dpl
                    val = plsc.load_gather(blk_v.at[b], [rowvs[bg], colv])
                    plsc.store_scatter(
                        t_v.at[b],
                        [dgv, msplat[bg * L // 128], drv, lanevs[bg % 8]],
                        val,
                    )

        g_start(0, 0)
        g_start(1, 1)

        @pl.loop(0, steps, step=NBUF)
        def _(c):
            for u in range(NBUF):
                b = u
                j = c + u
                g_wait(j, b)

                @pl.when(j >= NBUF)
                def _(j=j, b=b):
                    o_wait(j - NBUF, b)

                tec_chunk(b)
                o_start(j, b)

                @pl.when(j + NBUF < steps)
                def _(j=j, b=b):
                    g_start(j + NBUF, b)

        o_wait(steps - 2, 0)
        o_wait(steps - 1, 1)

    return k


def kernel(x, W):
    batch, seq = x.shape
    v = W.shape[0]
    W2 = _w2_kernel(v)(W.T)
    idx = jnp.transpose(x).reshape(-1).astype(jnp.int32)
    lp = P.bit_length() - 1
    j = ((idx >> (lp + 1)) << lp) | (idx & (P - 1))
    off = ((idx >> lp) & 1) << 6
    out5 = _emb_kernel(idx.shape[0], seq, batch)(j, off, W2)
    t = jnp.transpose(out5, (2, 4, 0, 1, 3))
    return t.reshape(batch, seq, D)


# speedup vs baseline: 4.2691x; 1.0049x over previous
import dataclasses

import jax
import jax.numpy as jnp
from jax import lax
from jax.experimental import pallas as pl
from jax.experimental.pallas import tpu as pltpu
from jax.experimental.pallas import tpu_sc as plsc

D = 64
L = 16
NC = 2
NS = 16
NW = NC * NS

P = 512
CHUNK = 256
NBUF = 2


def _w2_kernel(v):
    gb = 32

    def body(in_ref, out_ref):
        for q in range(gb):
            out_ref[q * P : (q + 1) * P, 0:D] = (
                jnp.swapaxes(in_ref[:, 2 * q * P : (2 * q + 1) * P], 0, 1) * 8.0
            )
            out_ref[q * P : (q + 1) * P, D : 2 * D] = (
                jnp.swapaxes(in_ref[:, (2 * q + 1) * P : (2 * q + 2) * P], 0, 1) * 8.0
            )

    g = pl.cdiv(v, 2 * gb * P)
    return pl.pallas_call(
        body,
        grid=(g,),
        in_specs=[pl.BlockSpec((D, 2 * gb * P), lambda i: (0, i))],
        out_specs=pl.BlockSpec((gb * P, 2 * D), lambda i: (i, 0)),
        out_shape=jax.ShapeDtypeStruct((g * gb * P, 2 * D), jnp.float32),
    )


def _emb_kernel(n_total: int, seq: int, batch: int):
    b_per_w = n_total // NW
    steps = b_per_w // CHUNK
    assert n_total == NW * CHUNK * steps and steps % NBUF == 0
    assert batch % CHUNK == 0 and b_per_w % CHUNK == 0
    n_m = batch // 128
    cm = CHUNK // 128
    mesh = plsc.VectorSubcoreMesh(core_axis_name="c", subcore_axis_name="s")

    @pl.kernel(
        out_type=jax.ShapeDtypeStruct((seq, D // 8, n_m, 8, 128), jnp.float32),
        mesh=mesh,
        compiler_params=dataclasses.replace(
            pltpu.CompilerParams(use_tc_tiling_on_sc=False),
            needs_layout_passes=False,
        ),
        scratch_types=[
            pltpu.VMEM((b_per_w,), jnp.int32),
            pltpu.VMEM((NBUF, CHUNK, 2 * D), jnp.float32),
            pltpu.VMEM((NBUF, CHUNK), jnp.int32),
            pltpu.VMEM((NBUF, D // 8, cm, 8, 128), jnp.float32),
        ]
        + [pltpu.SemaphoreType.DMA] * (3 * NBUF),
    )
    def k(idx2_hbm, off_hbm, w2_hbm, out_hbm, idx2_v, blk_v, off_v, t_v, *sems):
        sg = sems[:NBUF]
        sf = sems[NBUF : 2 * NBUF]
        so = sems[2 * NBUF :]
        wid = lax.axis_index("s") * NC + lax.axis_index("c")
        base = wid * b_per_w
        pltpu.sync_copy(idx2_hbm.at[pl.ds(base, b_per_w)], idx2_v)

        def g_start(i, b):
            pltpu.async_copy(
                w2_hbm.at[idx2_v.at[pl.ds(i * CHUNK, CHUNK)]], blk_v.at[b], sg[b]
            )
            pltpu.async_copy(
                off_hbm.at[pl.ds(base + i * CHUNK, CHUNK)], off_v.at[b], sf[b]
            )

        def g_wait(i, b):
            pltpu.make_async_copy(
                w2_hbm.at[idx2_v.at[pl.ds(i * CHUNK, CHUNK)]], blk_v.at[b], sg[b]
            ).wait()
            pltpu.make_async_copy(
                off_hbm.at[pl.ds(base + i * CHUNK, CHUNK)], off_v.at[b], sf[b]
            ).wait()

        lb = batch.bit_length() - 1

        def o_dst(i):
            j0 = base + i * CHUNK
            s = lax.shift_right_logical(j0, lb)
            m0 = lax.shift_right_logical(jnp.bitwise_and(j0, batch - 1), 7)
            return out_hbm.at[s, :, pl.ds(m0, cm)]

        def o_start(i, b):
            pltpu.async_copy(t_v.at[b], o_dst(i), so[b])

        def o_wait(i, b):
            pltpu.make_async_copy(t_v.at[b], o_dst(i), so[b]).wait()

        iota = lax.iota(jnp.int32, L)

        def tec_chunk(b):
            rowvs = [iota + (bg * L) for bg in range(CHUNK // L)]
            lanevs = [jnp.bitwise_and(r, 127) for r in rowvs[:8]]
            msplat = [jnp.full((L,), m, jnp.int32) for m in range(cm)]
            parvs = [off_v[b, pl.ds(bg * L, L)] for bg in range(CHUNK // L)]

            @plsc.parallel_loop(0, D, unroll=4)
            def _(d):
                dpl = jnp.bitwise_and(iota + d, D - 1)
                dgv = lax.shift_right_logical(dpl, 3)
                drv = jnp.bitwise_and(dpl, 7)
                for bg in range(CHUNK // L):
                    colv = parvs[bg] + dpl
                    val = plsc.load_gather(blk_v.at[b], [rowvs[bg], colv])
                    plsc.store_scatter(
                        t_v.at[b],
                        [dgv, msplat[bg * L // 128], drv, lanevs[bg % 8]],
                        val,
                    )

        g_start(0, 0)
        g_start(1, 1)

        @pl.loop(0, steps, step=NBUF)
        def _(c):
            for u in range(NBUF):
                b = u
                j = c + u
                g_wait(j, b)

                @pl.when(j >= NBUF)
                def _(j=j, b=b):
                    o_wait(j - NBUF, b)

                tec_chunk(b)
                o_start(j, b)

                @pl.when(j + NBUF < steps)
                def _(j=j, b=b):
                    g_start(j + NBUF, b)

        o_wait(steps - 2, 0)
        o_wait(steps - 1, 1)

    return k


def kernel(x, W):
    batch, seq = x.shape
    v = W.shape[0]
    W2 = _w2_kernel(v)(W.T)
    idx = jnp.transpose(x).reshape(-1).astype(jnp.int32)
    lp = P.bit_length() - 1
    j = ((idx >> (lp + 1)) << lp) | (idx & (P - 1))
    off = ((idx >> lp) & 1) << 6
    out5 = _emb_kernel(idx.shape[0], seq, batch)(j, off, W2)
    t = jnp.transpose(out5, (2, 4, 0, 1, 3))
    return t.reshape(batch, seq, D)
